# baseline (speedup 1.0000x reference)
import jax
import jax.numpy as jnp
from jax import lax
from jax.experimental import pallas as pl
from jax.experimental.pallas import tpu as pltpu
from jax.experimental.pallas import tpu_sc as plsc

_B = 16384
_D_DRV, _D_WK, _D_TM = 16, 3, 8
_D_OUT = _D_DRV + _D_WK + _D_TM + 1
_V_WK, _V_TM = 7, 1440
_V_DRV = 1000000

_NC, _NS = 2, 16
_NW = _NC * _NS
_BPW = _B // _NW
_L = 16
_NCH = _BPW // _L
_NE = _BPW * _D_DRV
_GCH = 128
_NG = _NE // _GCH


def _attr_body(drv_hbm, wk_hbm, tm_hbm, dist_hbm, wd_hbm, ww_hbm, wt_hbm,
               out_hbm, didx_v, eidx_v, widx_v, tidx_v, dist_v, drows_v,
               wtab_v, ttab_v, out_v, sem):
  wid = lax.axis_index("s") * _NC + lax.axis_index("c")
  base = wid * _BPW

  pltpu.sync_copy(drv_hbm.at[pl.ds(base, _BPW)], didx_v)

  def mk_eidx(ch, carry):
    r = didx_v[pl.ds(ch * _L, _L)]
    for c in range(_D_DRV):
      eidx_v[pl.ds(c * _BPW + ch * _L, _L)] = r + c * _V_DRV
    return carry

  lax.fori_loop(0, _NCH, mk_eidx, 0)

  copies = [
      pltpu.async_copy(wd_hbm.at[eidx_v.at[pl.ds(g * _GCH, _GCH)]],
                       drows_v.at[pl.ds(g * _GCH, _GCH)], sem)
      for g in range(_NG)
  ]

  pltpu.sync_copy(wk_hbm.at[pl.ds(base, _BPW)], widx_v)
  pltpu.sync_copy(tm_hbm.at[pl.ds(base, _BPW)], tidx_v)
  pltpu.sync_copy(dist_hbm.at[pl.ds(base, _BPW)], dist_v)
  pltpu.sync_copy(ww_hbm, wtab_v)
  pltpu.sync_copy(wt_hbm, ttab_v)

  def tail_chunk(ch, carry):
    rows = ch * _L + lax.iota(jnp.int32, _L)
    obase = rows * _D_OUT
    widx = widx_v[pl.ds(ch * _L, _L)] * _D_WK
    tidx = tidx_v[pl.ds(ch * _L, _L)] * _D_TM
    d = dist_v[pl.ds(ch * _L, _L)]
    for j in range(_D_WK):
      v = plsc.load_gather(wtab_v, [widx + j])
      plsc.store_scatter(out_v, [obase + (_D_DRV + j)], v)
    for j in range(_D_TM):
      v = plsc.load_gather(ttab_v, [tidx + j])
      plsc.store_scatter(out_v, [obase + (_D_DRV + _D_WK + j)], v)
    dn = ((d - 10.0) / 5.0 - 10.0) / 5.0
    plsc.store_scatter(out_v, [obase + (_D_OUT - 1)], dn)
    return carry

  lax.fori_loop(0, _NCH, tail_chunk, 0)

  for c in copies:
    c.wait()

  def drv_chunk(ch, carry):
    rows = ch * _L + lax.iota(jnp.int32, _L)
    obase = rows * _D_OUT
    for c in range(_D_DRV):
      v = drows_v[pl.ds(c * _BPW + ch * _L, _L)]
      plsc.store_scatter(out_v, [obase + c], v)
    return carry

  lax.fori_loop(0, _NCH, drv_chunk, 0)

  pltpu.sync_copy(out_v, out_hbm.at[pl.ds(base * _D_OUT, _BPW * _D_OUT)])


def _build_kernel():
  return pl.kernel(
      _attr_body,
      out_type=jax.ShapeDtypeStruct((_B * _D_OUT,), jnp.float32),
      mesh=plsc.VectorSubcoreMesh(core_axis_name="c", subcore_axis_name="s"),
      compiler_params=pltpu.CompilerParams(needs_layout_passes=False),
      scratch_types=[
          pltpu.VMEM((_BPW,), jnp.int32),
          pltpu.VMEM((_NE,), jnp.int32),
          pltpu.VMEM((_BPW,), jnp.int32),
          pltpu.VMEM((_BPW,), jnp.int32),
          pltpu.VMEM((_BPW,), jnp.float32),
          pltpu.VMEM((_NE,), jnp.float32),
          pltpu.VMEM((_V_WK * _D_WK,), jnp.float32),
          pltpu.VMEM((_V_TM * _D_TM,), jnp.float32),
          pltpu.VMEM((_BPW * _D_OUT,), jnp.float32),
          pltpu.SemaphoreType.DMA,
      ],
  )


def kernel(driverID, weekID, timeID, dist, W_driver, W_week, W_time):
  drv = driverID.reshape(_B).astype(jnp.int32)
  wk = weekID.reshape(_B).astype(jnp.int32)
  tm = timeID.reshape(_B).astype(jnp.int32)
  d = dist.reshape(_B).astype(jnp.float32)
  wd = W_driver.T.reshape(_V_DRV * _D_DRV)
  ww = W_week.reshape(_V_WK * _D_WK)
  wt = W_time.reshape(_V_TM * _D_TM)
  out = _build_kernel()(drv, wk, tm, d, wd, ww, wt)
  return out.reshape(_B, _D_OUT)


# speedup vs baseline: 11.2682x; 11.2682x over previous
import jax
import jax.numpy as jnp
from jax import lax
from jax.experimental import pallas as pl
from jax.experimental.pallas import tpu as pltpu
from jax.experimental.pallas import tpu_sc as plsc

_B = 16384
_D_DRV, _D_WK, _D_TM = 16, 3, 8
_D_OUT = _D_DRV + _D_WK + _D_TM + 1
_V_WK, _V_TM = 7, 1440
_V_DRV = 1000000

_LANES = 128
_TAIL0 = (_V_DRV // _LANES - 1) * _LANES
_V_TAIL = _V_DRV - _TAIL0
_MAXRB = _V_DRV // _LANES - 2

_NC, _NS = 2, 16
_NW = _NC * _NS
_BPW = _B // _NW
_L = 16
_NCH = _BPW // _L


def _attr_body(drv_hbm, wk_hbm, tm_hbm, dist_hbm, wdt_hbm, wtl_hbm, ww_hbm,
               wt_hbm, out_hbm, didx_v, widx_v, tidx_v, dist_v, tail_v,
               wtab_v, ttab_v, out_v, sem, *wins):
  wid = lax.axis_index("s") * _NC + lax.axis_index("c")
  base = wid * _BPW

  pltpu.sync_copy(drv_hbm.at[pl.ds(base, _BPW)], didx_v)
  pltpu.sync_copy(wk_hbm.at[pl.ds(base, _BPW)], widx_v)
  pltpu.sync_copy(tm_hbm.at[pl.ds(base, _BPW)], tidx_v)
  pltpu.sync_copy(dist_hbm.at[pl.ds(base, _BPW)], dist_v)
  pltpu.sync_copy(wtl_hbm, tail_v)
  pltpu.sync_copy(ww_hbm, wtab_v)
  pltpu.sync_copy(wt_hbm, ttab_v)

  lanes = lax.iota(jnp.int32, _L)

  def drv_chunk(ch, carry):
    r = didx_v[pl.ds(ch * _L, _L)]
    rs = []
    for i in range(_L):
      ri = lax.reduce_max(jnp.where(lanes == i, r, 0), (0,))
      rs.append(ri)
    starts = [
        jnp.minimum(ri // _LANES, _MAXRB) * _LANES for ri in rs
    ]
    copies = [
        pltpu.async_copy(wdt_hbm.at[pl.ds(0, _D_DRV), pl.ds(starts[i], _LANES)],
                         wins[i], sem)
        for i in range(_L)
    ]
    has_tail = lax.reduce_max(r, (0,)) >= _TAIL0
    crow = lax.iota(jnp.int32, _L)
    for i in range(_L):
      copies[i].wait()
      col = jnp.minimum(rs[i] - starts[i], _LANES - 1)
      v = plsc.load_gather(wins[i], [crow, jnp.full((_L,), col, jnp.int32)])
      @pl.when(has_tail)
      def _():
        toff = jnp.clip(rs[i] - _TAIL0, 0, _V_TAIL - 1) * _D_DRV
        tv = plsc.load_gather(tail_v, [toff + crow])
        out_v[pl.ds((ch * _L + i) * _D_OUT, _D_DRV)] = jnp.where(
            jnp.full((_L,), rs[i] >= _TAIL0), tv, v)

      @pl.when(jnp.logical_not(has_tail))
      def _():
        out_v[pl.ds((ch * _L + i) * _D_OUT, _D_DRV)] = v
    return carry

  lax.fori_loop(0, _NCH, drv_chunk, 0)

  def tail_chunk(ch, carry):
    rows = ch * _L + lanes
    obase = rows * _D_OUT
    widx = widx_v[pl.ds(ch * _L, _L)] * _D_WK
    tidx = tidx_v[pl.ds(ch * _L, _L)] * _D_TM
    d = dist_v[pl.ds(ch * _L, _L)]
    for j in range(_D_WK):
      v = plsc.load_gather(wtab_v, [widx + j])
      plsc.store_scatter(out_v, [obase + (_D_DRV + j)], v)
    for j in range(_D_TM):
      v = plsc.load_gather(ttab_v, [tidx + j])
      plsc.store_scatter(out_v, [obase + (_D_DRV + _D_WK + j)], v)
    dn = ((d - 10.0) / 5.0 - 10.0) / 5.0
    plsc.store_scatter(out_v, [obase + (_D_OUT - 1)], dn)
    return carry

  lax.fori_loop(0, _NCH, tail_chunk, 0)

  pltpu.sync_copy(out_v, out_hbm.at[pl.ds(base * _D_OUT, _BPW * _D_OUT)])


def _build_kernel():
  return pl.kernel(
      _attr_body,
      out_type=jax.ShapeDtypeStruct((_B * _D_OUT,), jnp.float32),
      mesh=plsc.VectorSubcoreMesh(core_axis_name="c", subcore_axis_name="s"),
      compiler_params=pltpu.CompilerParams(needs_layout_passes=False),
      scratch_types=[
          pltpu.VMEM((_BPW,), jnp.int32),
          pltpu.VMEM((_BPW,), jnp.int32),
          pltpu.VMEM((_BPW,), jnp.int32),
          pltpu.VMEM((_BPW,), jnp.float32),
          pltpu.VMEM((_V_TAIL * _D_DRV,), jnp.float32),
          pltpu.VMEM((_V_WK * _D_WK,), jnp.float32),
          pltpu.VMEM((_V_TM * _D_TM,), jnp.float32),
          pltpu.VMEM((_BPW * _D_OUT,), jnp.float32),
          pltpu.SemaphoreType.DMA,
      ] + [pltpu.VMEM((_D_DRV, _LANES), jnp.float32) for _ in range(_L)],
  )


def kernel(driverID, weekID, timeID, dist, W_driver, W_week, W_time):
  drv = driverID.reshape(_B).astype(jnp.int32)
  wk = weekID.reshape(_B).astype(jnp.int32)
  tm = timeID.reshape(_B).astype(jnp.int32)
  d = dist.reshape(_B).astype(jnp.float32)
  wdt = W_driver.T
  wtl = W_driver[_TAIL0:].reshape(_V_TAIL * _D_DRV)
  ww = W_week.reshape(_V_WK * _D_WK)
  wt = W_time.reshape(_V_TM * _D_TM)
  out = _build_kernel()(drv, wk, tm, d, wdt, wtl, ww, wt)
  return out.reshape(_B, _D_OUT)


# speedup vs baseline: 11.5064x; 1.0211x over previous
import jax
import jax.numpy as jnp
from jax import lax
from jax.experimental import pallas as pl
from jax.experimental.pallas import tpu as pltpu
from jax.experimental.pallas import tpu_sc as plsc

_B = 16384
_D_DRV, _D_WK, _D_TM = 16, 3, 8
_D_OUT = _D_DRV + _D_WK + _D_TM + 1
_V_WK, _V_TM = 7, 1440
_V_DRV = 1000000

_LANES = 128
_TAIL0 = (_V_DRV // _LANES - 1) * _LANES
_V_TAIL = _V_DRV - _TAIL0
_MAXRB = _V_DRV // _LANES - 2

_NC, _NS = 2, 16
_NW = _NC * _NS
_BPW = _B // _NW
_L = 16
_NCH = _BPW // _L


def _attr_body(drv_hbm, wk_hbm, tm_hbm, dist_hbm, wdt_hbm, wtl_hbm, ww_hbm,
               wt_hbm, out_hbm, didx_v, widx_v, tidx_v, dist_v, tail_v,
               wtab_v, ttab_v, out_v, sem, *wins):
  wid = lax.axis_index("s") * _NC + lax.axis_index("c")
  base = wid * _BPW

  pltpu.sync_copy(drv_hbm.at[pl.ds(base, _BPW)], didx_v)
  pltpu.sync_copy(wk_hbm.at[pl.ds(base, _BPW)], widx_v)
  pltpu.sync_copy(tm_hbm.at[pl.ds(base, _BPW)], tidx_v)
  pltpu.sync_copy(dist_hbm.at[pl.ds(base, _BPW)], dist_v)
  pltpu.sync_copy(wtl_hbm, tail_v)
  pltpu.sync_copy(ww_hbm, wtab_v)
  pltpu.sync_copy(wt_hbm, ttab_v)

  lanes = lax.iota(jnp.int32, _L)

  def chunk(ch, carry):
    r = didx_v[pl.ds(ch * _L, _L)]
    rs = []
    for i in range(_L):
      ri = lax.reduce_max(jnp.where(lanes == i, r, 0), (0,))
      rs.append(ri)
    starts = [
        jnp.minimum(ri // _LANES, _MAXRB) * _LANES for ri in rs
    ]
    copies = [
        pltpu.async_copy(wdt_hbm.at[pl.ds(0, _D_DRV), pl.ds(starts[i], _LANES)],
                         wins[i], sem)
        for i in range(_L)
    ]

    rows = ch * _L + lanes
    obase = rows * _D_OUT
    widx = widx_v[pl.ds(ch * _L, _L)] * _D_WK
    tidx = tidx_v[pl.ds(ch * _L, _L)] * _D_TM
    d = dist_v[pl.ds(ch * _L, _L)]
    for j in range(_D_WK):
      v = plsc.load_gather(wtab_v, [widx + j])
      plsc.store_scatter(out_v, [obase + (_D_DRV + j)], v)
    for j in range(_D_TM):
      v = plsc.load_gather(ttab_v, [tidx + j])
      plsc.store_scatter(out_v, [obase + (_D_DRV + _D_WK + j)], v)
    dn = ((d - 10.0) / 5.0 - 10.0) / 5.0
    plsc.store_scatter(out_v, [obase + (_D_OUT - 1)], dn)

    has_tail = lax.reduce_max(r, (0,)) >= _TAIL0
    crow = lax.iota(jnp.int32, _L)
    for i in range(_L):
      copies[i].wait()
      col = jnp.minimum(rs[i] - starts[i], _LANES - 1)
      v = plsc.load_gather(wins[i], [crow, jnp.full((_L,), col, jnp.int32)])
      @pl.when(has_tail)
      def _():
        toff = jnp.clip(rs[i] - _TAIL0, 0, _V_TAIL - 1) * _D_DRV
        tv = plsc.load_gather(tail_v, [toff + crow])
        out_v[pl.ds((ch * _L + i) * _D_OUT, _D_DRV)] = jnp.where(
            jnp.full((_L,), rs[i] >= _TAIL0), tv, v)

      @pl.when(jnp.logical_not(has_tail))
      def _():
        out_v[pl.ds((ch * _L + i) * _D_OUT, _D_DRV)] = v
    return carry

  lax.fori_loop(0, _NCH, chunk, 0)

  pltpu.sync_copy(out_v, out_hbm.at[pl.ds(base * _D_OUT, _BPW * _D_OUT)])


def _build_kernel():
  return pl.kernel(
      _attr_body,
      out_type=jax.ShapeDtypeStruct((_B * _D_OUT,), jnp.float32),
      mesh=plsc.VectorSubcoreMesh(core_axis_name="c", subcore_axis_name="s"),
      compiler_params=pltpu.CompilerParams(needs_layout_passes=False),
      scratch_types=[
          pltpu.VMEM((_BPW,), jnp.int32),
          pltpu.VMEM((_BPW,), jnp.int32),
          pltpu.VMEM((_BPW,), jnp.int32),
          pltpu.VMEM((_BPW,), jnp.float32),
          pltpu.VMEM((_V_TAIL * _D_DRV,), jnp.float32),
          pltpu.VMEM((_V_WK * _D_WK,), jnp.float32),
          pltpu.VMEM((_V_TM * _D_TM,), jnp.float32),
          pltpu.VMEM((_BPW * _D_OUT,), jnp.float32),
          pltpu.SemaphoreType.DMA,
      ] + [pltpu.VMEM((_D_DRV, _LANES), jnp.float32) for _ in range(_L)],
  )


def kernel(driverID, weekID, timeID, dist, W_driver, W_week, W_time):
  drv = driverID.reshape(_B).astype(jnp.int32)
  wk = weekID.reshape(_B).astype(jnp.int32)
  tm = timeID.reshape(_B).astype(jnp.int32)
  d = dist.reshape(_B).astype(jnp.float32)
  wdt = W_driver.T
  wtl = W_driver[_TAIL0:].reshape(_V_TAIL * _D_DRV)
  ww = W_week.reshape(_V_WK * _D_WK)
  wt = W_time.reshape(_V_TM * _D_TM)
  out = _build_kernel()(drv, wk, tm, d, wdt, wtl, ww, wt)
  return out.reshape(_B, _D_OUT)


# speedup vs baseline: 12.4694x; 1.0837x over previous
import jax
import jax.numpy as jnp
from jax import lax
from jax.experimental import pallas as pl
from jax.experimental.pallas import tpu as pltpu
from jax.experimental.pallas import tpu_sc as plsc

_B = 16384
_D_DRV, _D_WK, _D_TM = 16, 3, 8
_D_OUT = _D_DRV + _D_WK + _D_TM + 1
_V_WK, _V_TM = 7, 1440
_V_DRV = 1000000

_LANES = 128
_TAIL0 = (_V_DRV // _LANES - 1) * _LANES
_V_TAIL = _V_DRV - _TAIL0
_MAXRB = _V_DRV // _LANES - 2

_NC, _NS = 2, 16
_NW = _NC * _NS
_BPW = _B // _NW
_L = 16
_NCH = _BPW // _L


def _attr_body(drv_hbm, wk_hbm, tm_hbm, dist_hbm, wdt_hbm, wtl_hbm, ww_hbm,
               wt_hbm, out_hbm, didx_v, widx_v, tidx_v, dist_v, tail_v,
               wtab_v, ttab_v, out_v, sem_a, sem_b, *wins):
  wid = lax.axis_index("s") * _NC + lax.axis_index("c")
  base = wid * _BPW

  pltpu.sync_copy(drv_hbm.at[pl.ds(base, _BPW)], didx_v)
  pltpu.sync_copy(wk_hbm.at[pl.ds(base, _BPW)], widx_v)
  pltpu.sync_copy(tm_hbm.at[pl.ds(base, _BPW)], tidx_v)
  pltpu.sync_copy(dist_hbm.at[pl.ds(base, _BPW)], dist_v)
  pltpu.sync_copy(wtl_hbm, tail_v)
  pltpu.sync_copy(ww_hbm, wtab_v)
  pltpu.sync_copy(wt_hbm, ttab_v)

  lanes = lax.iota(jnp.int32, _L)
  wins_a, wins_b = wins[:_L], wins[_L:]

  def id_scalars(ch):
    r = didx_v[pl.ds(ch * _L, _L)]
    rs, starts = [], []
    for i in range(_L):
      ri = lax.reduce_max(jnp.where(lanes == i, r, 0), (0,))
      rs.append(ri)
      starts.append(jnp.minimum(ri // _LANES, _MAXRB) * _LANES)
    return r, rs, starts

  def fire(ch, ws, sem):
    _, _, starts = id_scalars(ch)
    for i in range(_L):
      pltpu.async_copy(
          wdt_hbm.at[pl.ds(0, _D_DRV), pl.ds(starts[i], _LANES)], ws[i], sem)

  def drain_only(ws, sem):
    for i in range(_L):
      pltpu.make_async_copy(
          wdt_hbm.at[pl.ds(0, _D_DRV), pl.ds(0, _LANES)], ws[i], sem).wait()

  def tail_cols(ch):
    rows = ch * _L + lanes
    obase = rows * _D_OUT
    widx = widx_v[pl.ds(ch * _L, _L)] * _D_WK
    tidx = tidx_v[pl.ds(ch * _L, _L)] * _D_TM
    d = dist_v[pl.ds(ch * _L, _L)]
    for j in range(_D_WK):
      v = plsc.load_gather(wtab_v, [widx + j])
      plsc.store_scatter(out_v, [obase + (_D_DRV + j)], v)
    for j in range(_D_TM):
      v = plsc.load_gather(ttab_v, [tidx + j])
      plsc.store_scatter(out_v, [obase + (_D_DRV + _D_WK + j)], v)
    dn = ((d - 10.0) / 5.0 - 10.0) / 5.0
    plsc.store_scatter(out_v, [obase + (_D_OUT - 1)], dn)

  def extract(ch, ws, sem):
    tail_cols(ch)
    r, rs, starts = id_scalars(ch)
    has_tail = lax.reduce_max(r, (0,)) >= _TAIL0
    crow = lax.iota(jnp.int32, _L)
    for i in range(_L):
      pltpu.make_async_copy(
          wdt_hbm.at[pl.ds(0, _D_DRV), pl.ds(0, _LANES)], ws[i], sem).wait()
      col = jnp.minimum(rs[i] - starts[i], _LANES - 1)
      v = plsc.load_gather(ws[i], [crow, jnp.full((_L,), col, jnp.int32)])
      @pl.when(has_tail)
      def _():
        toff = jnp.clip(rs[i] - _TAIL0, 0, _V_TAIL - 1) * _D_DRV
        tv = plsc.load_gather(tail_v, [toff + crow])
        out_v[pl.ds((ch * _L + i) * _D_OUT, _D_DRV)] = jnp.where(
            jnp.full((_L,), rs[i] >= _TAIL0), tv, v)

      @pl.when(jnp.logical_not(has_tail))
      def _():
        out_v[pl.ds((ch * _L + i) * _D_OUT, _D_DRV)] = v

  fire(0, wins_a, sem_a)

  def pipe(g, carry):
    fire(2 * g + 1, wins_b, sem_b)
    extract(2 * g, wins_a, sem_a)
    fire(jnp.minimum(2 * g + 2, _NCH - 1), wins_a, sem_a)
    extract(2 * g + 1, wins_b, sem_b)
    return carry

  lax.fori_loop(0, _NCH // 2, pipe, 0)
  drain_only(wins_a, sem_a)

  pltpu.sync_copy(out_v, out_hbm.at[pl.ds(base * _D_OUT, _BPW * _D_OUT)])


def _build_kernel():
  return pl.kernel(
      _attr_body,
      out_type=jax.ShapeDtypeStruct((_B * _D_OUT,), jnp.float32),
      mesh=plsc.VectorSubcoreMesh(core_axis_name="c", subcore_axis_name="s"),
      compiler_params=pltpu.CompilerParams(needs_layout_passes=False),
      scratch_types=[
          pltpu.VMEM((_BPW,), jnp.int32),
          pltpu.VMEM((_BPW,), jnp.int32),
          pltpu.VMEM((_BPW,), jnp.int32),
          pltpu.VMEM((_BPW,), jnp.float32),
          pltpu.VMEM((_V_TAIL * _D_DRV,), jnp.float32),
          pltpu.VMEM((_V_WK * _D_WK,), jnp.float32),
          pltpu.VMEM((_V_TM * _D_TM,), jnp.float32),
          pltpu.VMEM((_BPW * _D_OUT,), jnp.float32),
          pltpu.SemaphoreType.DMA,
          pltpu.SemaphoreType.DMA,
      ] + [pltpu.VMEM((_D_DRV, _LANES), jnp.float32) for _ in range(2 * _L)],
  )


def kernel(driverID, weekID, timeID, dist, W_driver, W_week, W_time):
  drv = driverID.reshape(_B).astype(jnp.int32)
  wk = weekID.reshape(_B).astype(jnp.int32)
  tm = timeID.reshape(_B).astype(jnp.int32)
  d = dist.reshape(_B).astype(jnp.float32)
  wdt = W_driver.T
  wtl = W_driver[_TAIL0:].reshape(_V_TAIL * _D_DRV)
  ww = W_week.reshape(_V_WK * _D_WK)
  wt = W_time.reshape(_V_TM * _D_TM)
  out = _build_kernel()(drv, wk, tm, d, wdt, wtl, ww, wt)
  return out.reshape(_B, _D_OUT)


# speedup vs baseline: 15.3437x; 1.2305x over previous
import jax
import jax.numpy as jnp
from jax import lax
from jax.experimental import pallas as pl
from jax.experimental.pallas import tpu as pltpu
from jax.experimental.pallas import tpu_sc as plsc

_B = 16384
_D_DRV, _D_WK, _D_TM = 16, 3, 8
_D_OUT = _D_DRV + _D_WK + _D_TM + 1
_V_WK, _V_TM = 7, 1440
_V_DRV = 1000000

_LANES = 128
_TAIL0 = (_V_DRV // _LANES - 1) * _LANES
_V_TAIL = _V_DRV - _TAIL0
_MAXRB = _V_DRV // _LANES - 2

_NC, _NS = 2, 16
_NW = _NC * _NS
_BPW = _B // _NW
_L = 16
_NCH = _BPW // _L


def _attr_body(drv_hbm, wk_hbm, tm_hbm, dist_hbm, wdt_hbm, wtl_hbm, ww_hbm,
               wt_hbm, out_hbm, didx_v, widx_v, tidx_v, dist_v, tail_v,
               wtab_v, ttab_v, out_v, sem_a, sem_b, *wins):
  wid = lax.axis_index("s") * _NC + lax.axis_index("c")
  base = wid * _BPW

  pltpu.sync_copy(drv_hbm.at[pl.ds(base, _BPW)], didx_v)
  pltpu.sync_copy(wk_hbm.at[pl.ds(base, _BPW)], widx_v)
  pltpu.sync_copy(tm_hbm.at[pl.ds(base, _BPW)], tidx_v)
  pltpu.sync_copy(dist_hbm.at[pl.ds(base, _BPW)], dist_v)
  pltpu.sync_copy(wtl_hbm, tail_v)
  pltpu.sync_copy(ww_hbm, wtab_v)
  pltpu.sync_copy(wt_hbm, ttab_v)

  lanes = lax.iota(jnp.int32, _L)
  wins_a, wins_b = wins[:_L], wins[_L:]

  def id_scalars(ch):
    r = didx_v[pl.ds(ch * _L, _L)]
    rs, starts = [], []
    for i in range(_L):
      ri = lax.reduce_max(jnp.where(lanes == i, r, 0), (0,))
      rs.append(ri)
      starts.append(jnp.minimum(ri // _LANES, _MAXRB) * _LANES)
    return r, rs, starts

  def fire(ch, ws, sem):
    _, _, starts = id_scalars(ch)
    for i in range(_L):
      pltpu.async_copy(
          wdt_hbm.at[pl.ds(0, _D_DRV), pl.ds(starts[i], _LANES)], ws[i], sem)

  def drain_only(ws, sem):
    for i in range(_L):
      pltpu.make_async_copy(
          wdt_hbm.at[pl.ds(0, _D_DRV), pl.ds(0, _LANES)], ws[i], sem).wait()

  def tail_cols(ch):
    rows = ch * _L + lanes
    widx = widx_v[pl.ds(ch * _L, _L)] * _D_WK
    tidx = tidx_v[pl.ds(ch * _L, _L)]
    d = dist_v[pl.ds(ch * _L, _L)]
    for j in range(_D_WK):
      v = plsc.load_gather(wtab_v, [widx + j])
      plsc.store_scatter(out_v, [jnp.full((_L,), _D_DRV + j, jnp.int32), rows], v)
    for j in range(_D_TM):
      v = plsc.load_gather(ttab_v, [jnp.full((_L,), j, jnp.int32), tidx])
      plsc.store_scatter(out_v, [jnp.full((_L,), _D_DRV + _D_WK + j, jnp.int32), rows], v)
    dn = ((d - 10.0) / 5.0 - 10.0) / 5.0
    plsc.store_scatter(out_v, [jnp.full((_L,), _D_OUT - 1, jnp.int32), rows], dn)

  def extract(ch, ws, sem):
    tail_cols(ch)
    r, rs, starts = id_scalars(ch)
    has_tail = lax.reduce_max(r, (0,)) >= _TAIL0
    crow = lax.iota(jnp.int32, _L)
    for i in range(_L):
      pltpu.make_async_copy(
          wdt_hbm.at[pl.ds(0, _D_DRV), pl.ds(0, _LANES)], ws[i], sem).wait()
      col = jnp.minimum(rs[i] - starts[i], _LANES - 1)
      v = plsc.load_gather(ws[i], [crow, jnp.full((_L,), col, jnp.int32)])
      ocol = jnp.full((_L,), ch * _L + i, jnp.int32)
      @pl.when(has_tail)
      def _():
        toff = jnp.clip(rs[i] - _TAIL0, 0, _V_TAIL - 1) * _D_DRV
        tv = plsc.load_gather(tail_v, [toff + crow])
        plsc.store_scatter(out_v, [crow, ocol], jnp.where(
            jnp.full((_L,), rs[i] >= _TAIL0), tv, v))

      @pl.when(jnp.logical_not(has_tail))
      def _():
        plsc.store_scatter(out_v, [crow, ocol], v)

  fire(0, wins_a, sem_a)

  def pipe(g, carry):
    fire(2 * g + 1, wins_b, sem_b)
    extract(2 * g, wins_a, sem_a)
    fire(jnp.minimum(2 * g + 2, _NCH - 1), wins_a, sem_a)
    extract(2 * g + 1, wins_b, sem_b)
    return carry

  lax.fori_loop(0, _NCH // 2, pipe, 0)
  drain_only(wins_a, sem_a)

  pltpu.sync_copy(out_v, out_hbm.at[pl.ds(0, _D_OUT), pl.ds(base, _BPW)])


def _build_kernel():
  return pl.kernel(
      _attr_body,
      out_type=jax.ShapeDtypeStruct((_D_OUT, _B), jnp.float32),
      mesh=plsc.VectorSubcoreMesh(core_axis_name="c", subcore_axis_name="s"),
      compiler_params=pltpu.CompilerParams(needs_layout_passes=False),
      scratch_types=[
          pltpu.VMEM((_BPW,), jnp.int32),
          pltpu.VMEM((_BPW,), jnp.int32),
          pltpu.VMEM((_BPW,), jnp.int32),
          pltpu.VMEM((_BPW,), jnp.float32),
          pltpu.VMEM((_V_TAIL * _D_DRV,), jnp.float32),
          pltpu.VMEM((_V_WK * _D_WK,), jnp.float32),
          pltpu.VMEM((_D_TM, _V_TM), jnp.float32),
          pltpu.VMEM((_D_OUT, _BPW), jnp.float32),
          pltpu.SemaphoreType.DMA,
          pltpu.SemaphoreType.DMA,
      ] + [pltpu.VMEM((_D_DRV, _LANES), jnp.float32) for _ in range(2 * _L)],
  )


def kernel(driverID, weekID, timeID, dist, W_driver, W_week, W_time):
  drv = driverID.reshape(_B).astype(jnp.int32)
  wk = weekID.reshape(_B).astype(jnp.int32)
  tm = timeID.reshape(_B).astype(jnp.int32)
  d = dist.reshape(_B).astype(jnp.float32)
  wdt = W_driver.T
  wtl = W_driver[_TAIL0:].reshape(_V_TAIL * _D_DRV)
  ww = W_week.reshape(_V_WK * _D_WK)
  wt = W_time.T
  out = _build_kernel()(drv, wk, tm, d, wdt, wtl, ww, wt)
  return out.T
